# baseline (speedup 1.0000x reference)
import functools

import jax
import jax.numpy as jnp
from jax import lax
from jax.experimental import pallas as pl
from jax.experimental.pallas import tpu as pltpu
from jax.experimental.pallas import tpu_sc as plsc

N = 10000
E = 160000
HD = 256
ED = 128

_BN = 2000
_BE = 1280
_EA = 81920
_EB = E - _EA

_NC, _NS = 2, 16
_NW = _NC * _NS
_NP = 10240
_GC = 40
_GNB = 4
_SC_CH = 80


def _sc_mesh():
    return plsc.VectorSubcoreMesh(core_axis_name="c", subcore_axis_name="s")


def _gather_body(nidx, per_w, nch, tail, nb_, *refs):
    tbls = refs[:nidx]
    idx_hbms = refs[nidx:2 * nidx]
    outs = refs[2 * nidx:3 * nidx]
    sc = refs[3 * nidx:]
    nb = nb_
    idx_vs = sc[:nidx]
    bufs = sc[nidx:nidx + nb * nidx]
    gsems = sc[nidx + nb * nidx:nidx + 2 * nb * nidx]
    wsems = sc[nidx + 2 * nb * nidx:nidx + 3 * nb * nidx]

    wid = lax.axis_index("s") * _NC + lax.axis_index("c")
    base = pl.multiple_of(wid * per_w, 8)

    for a in range(nidx):
        pltpu.sync_copy(idx_hbms[a].at[pl.ds(base, per_w)], idx_vs[a])

    def g_start(a, ch, b, sz=_GC):
        off = pl.multiple_of(ch * _GC, 8)
        pltpu.async_copy(tbls[a].at[idx_vs[a].at[pl.ds(off, sz)]],
                         bufs[nb * a + b].at[pl.ds(0, sz)],
                         gsems[nb * a + b])

    def g_wait(a, b, sz=_GC):
        pltpu.make_async_copy(tbls[a].at[idx_vs[a].at[pl.ds(0, sz)]],
                              bufs[nb * a + b].at[pl.ds(0, sz)],
                              gsems[nb * a + b]).wait()

    def w_start(a, ch, b, sz=_GC):
        pltpu.async_copy(bufs[nb * a + b].at[pl.ds(0, sz)],
                         outs[a].at[pl.ds(base + ch * _GC, sz)],
                         wsems[nb * a + b])

    def w_wait(a, b, sz=_GC):
        pltpu.make_async_copy(bufs[nb * a + b].at[pl.ds(0, sz)],
                              outs[a].at[pl.ds(0, sz)],
                              wsems[nb * a + b]).wait()

    lead = min(nb - 1, max(1, nch - 1))

    def step(c2, b, static):
        bg = (b + lead) % nb
        for a in range(nidx):
            if static:
                if c2 + lead < nch:
                    if c2 + lead - nb >= 0:
                        w_wait(a, bg)
                    g_start(a, c2 + lead, bg)
            else:
                @pl.when(c2 + lead < nch)
                def _():
                    @pl.when(c2 + lead - nb >= 0)
                    def _():
                        w_wait(a, bg)
                    g_start(a, c2 + lead, bg)
        for a in range(nidx):
            g_wait(a, b)
            w_start(a, c2, b)

    for a in range(nidx):
        for b in range(lead):
            g_start(a, b, b)

    nfull = (nch // nb) * nb

    @pl.loop(0, nfull, step=nb)
    def _(ch0):
        for b in range(nb):
            step(ch0 + b, b, static=False)

    for c2 in range(nfull, nch):
        step(c2, c2 % nb, static=True)

    for c2 in range(max(0, nch - nb), nch):
        for a in range(nidx):
            w_wait(a, c2 % nb)

    if tail:
        for a in range(nidx):
            g_start(a, nch, 0, tail)
        for a in range(nidx):
            g_wait(a, 0, tail)
            w_start(a, nch, 0, tail)
        for a in range(nidx):
            w_wait(a, 0, tail)


def _sc_gather(pairs):
    nidx = len(pairs)
    el = pairs[0][1].shape[0]
    per_w = el // _NW
    assert per_w * _NW == el and per_w % 8 == 0
    nch = per_w // _GC
    tail = per_w % _GC
    assert tail % 8 == 0
    nb = 8 if nidx == 1 else 4
    scratch = []
    scratch += [pltpu.VMEM((per_w,), jnp.int32) for _ in range(nidx)]
    for t, _ in pairs:
        scratch += [pltpu.VMEM((_GC, t.shape[1]), jnp.float32)
                    for _ in range(nb)]
    scratch += [pltpu.SemaphoreType.DMA for _ in range(2 * nb * nidx)]
    fn = pl.kernel(
        functools.partial(_gather_body, nidx, per_w, nch, tail, nb),
        out_type=tuple(jax.ShapeDtypeStruct((el, t.shape[1]), jnp.float32)
                       for t, _ in pairs),
        mesh=_sc_mesh(),
        scratch_types=scratch,
    )
    return fn(*[t for t, _ in pairs], *[i for _, i in pairs])


def _scatter_pipe(nch, sid, msg_hbm, out_hbm, shared, idx_v, mb, lsems,
                  ssems):
    rows0 = pl.multiple_of(sid * (_NP // _NS), 8)
    ebase = sid * (nch * _SC_CH)

    for j in range(8):
        pltpu.sync_copy(mb[0], shared.at[pl.ds(rows0 + j * _SC_CH, _SC_CH)])
    plsc.subcore_barrier()

    nb = len(mb)
    lead = 1

    def l_start(ch, b):
        pltpu.async_copy(msg_hbm.at[pl.ds(ebase + ch * _SC_CH, _SC_CH)],
                         mb[b], lsems[b])

    def l_wait(b):
        pltpu.make_async_copy(msg_hbm.at[pl.ds(0, _SC_CH)], mb[b],
                              lsems[b]).wait()

    def s_start(ch, b):
        pltpu.async_copy(mb[b], shared.at[idx_v.at[ch]], ssems[b], add=True)

    def s_wait(b):
        pltpu.make_async_copy(mb[b], shared.at[idx_v.at[0]],
                              ssems[b]).wait()

    def step(c2, b, static):
        bg = (b + lead) % nb
        if static:
            if c2 + lead < nch:
                if c2 + lead - nb >= 0:
                    s_wait(bg)
                l_start(c2 + lead, bg)
        else:
            @pl.when(c2 + lead < nch)
            def _():
                @pl.when(c2 + lead - nb >= 0)
                def _():
                    s_wait(bg)
                l_start(c2 + lead, bg)
        l_wait(b)
        s_start(c2, b)

    for b in range(lead):
        l_start(b, b)

    nfull = (nch // nb) * nb

    @pl.loop(0, nfull, step=nb)
    def _(ch0):
        for b in range(nb):
            step(ch0 + b, b, static=False)

    for c2 in range(nfull, nch):
        step(c2, c2 % nb, static=True)

    for c2 in range(max(0, nch - nb), nch):
        s_wait(c2 % nb)

    plsc.subcore_barrier()

    for j in range(8):
        b = j % 2
        pltpu.sync_copy(shared.at[pl.ds(rows0 + j * _SC_CH, _SC_CH)], mb[b])
        pltpu.sync_copy(mb[b], out_hbm.at[pl.ds(rows0 + j * _SC_CH, _SC_CH)])


def _scatter_body(nch, msg0, msg1, srcr, zeros_hbm, out0, out1,
                  shared, idx_v, mb0, mb1, mb2,
                  lsem0, lsem1, lsem2, ssem0, ssem1, ssem2):
    cid = lax.axis_index("c")
    sid = lax.axis_index("s")
    pltpu.sync_copy(srcr.at[sid], idx_v)
    pltpu.sync_copy(zeros_hbm, mb0)

    @pl.when(cid == 0)
    def _():
        _scatter_pipe(nch, sid, msg0, out0, shared, idx_v, (mb0, mb1, mb2),
                      (lsem0, lsem1, lsem2), (ssem0, ssem1, ssem2))

    @pl.when(cid == 1)
    def _():
        _scatter_pipe(nch, sid, msg1, out1, shared, idx_v, (mb0, mb1, mb2),
                      (lsem0, lsem1, lsem2), (ssem0, ssem1, ssem2))


def _sc_scatter_add(msg0, msg1, srcr, zeros):
    nch = srcr.shape[1]
    fn = pl.kernel(
        functools.partial(_scatter_body, nch),
        out_type=(jax.ShapeDtypeStruct((_NP, ED), jnp.float32),
                  jax.ShapeDtypeStruct((_NP, ED), jnp.float32)),
        mesh=_sc_mesh(),
        scratch_types=[
            pltpu.VMEM_SHARED((_NP, ED), jnp.float32),
            pltpu.VMEM((nch, _SC_CH), jnp.int32),
            pltpu.VMEM((_SC_CH, ED), jnp.float32),
            pltpu.VMEM((_SC_CH, ED), jnp.float32),
            pltpu.VMEM((_SC_CH, ED), jnp.float32),
            pltpu.SemaphoreType.DMA,
            pltpu.SemaphoreType.DMA,
            pltpu.SemaphoreType.DMA,
            pltpu.SemaphoreType.DMA,
            pltpu.SemaphoreType.DMA,
            pltpu.SemaphoreType.DMA,
        ],
    )
    return fn(msg0, msg1, srcr, zeros)


def _lrelu(x):
    return jnp.where(x > 0, x, 0.2 * x)


def _mlp2_body(n_in, act, ln, residual, nout, *refs):
    xs = refs[:n_in]
    ws = refs[n_in:2 * n_in]
    b1 = refs[2 * n_in]
    w2 = refs[2 * n_in + 1]
    b2 = refs[2 * n_in + 2]
    k = 2 * n_in + 3
    if ln:
        g_ref, bb_ref = refs[k], refs[k + 1]
        k += 2
    if residual:
        res_ref = refs[k]
        k += 1
    out_refs = refs[k:k + nout]

    acc = b1[...].astype(jnp.float32)
    for x_ref, w_ref in zip(xs, ws):
        acc = acc + jnp.dot(x_ref[...], w_ref[...],
                            preferred_element_type=jnp.float32)
    y = act(acc)
    out = jnp.dot(y, w2[...], preferred_element_type=jnp.float32) + b2[...]
    if ln:
        m = jnp.mean(out, axis=-1, keepdims=True)
        v = jnp.mean((out - m) ** 2, axis=-1, keepdims=True)
        out = (out - m) * lax.rsqrt(v + 1e-5) * g_ref[...] + bb_ref[...]
    if residual:
        out = out + res_ref[...]
    if nout == 1:
        out_refs[0][...] = out
    else:
        off = 0
        for o_ref in out_refs:
            w = o_ref.shape[1]
            o_ref[...] = out[:, off:off + w]
            off += w


def _mlp2(xs, w1s, b1, w2, b2, *, act=_lrelu, ln=None, res=None,
          block_rows=_BE, out_split=None):
    rows = xs[0].shape[0]
    assert rows % block_rows == 0
    out_dim = w2.shape[1]
    n_in = len(xs)
    grid = (rows // block_rows,)
    widths = out_split if out_split is not None else (out_dim,)

    in_specs = [pl.BlockSpec((block_rows, x.shape[1]), lambda i: (i, 0))
                for x in xs]
    in_specs += [pl.BlockSpec(w.shape, lambda i: (0, 0)) for w in w1s]
    operands = list(xs) + list(w1s)
    b1r = b1.reshape(1, -1)
    b2r = b2.reshape(1, -1)
    in_specs += [pl.BlockSpec(b1r.shape, lambda i: (0, 0)),
                 pl.BlockSpec(w2.shape, lambda i: (0, 0)),
                 pl.BlockSpec(b2r.shape, lambda i: (0, 0))]
    operands += [b1r, w2, b2r]
    if ln is not None:
        g, bb = ln
        gr, bbr = g.reshape(1, -1), bb.reshape(1, -1)
        in_specs += [pl.BlockSpec(gr.shape, lambda i: (0, 0)),
                     pl.BlockSpec(bbr.shape, lambda i: (0, 0))]
        operands += [gr, bbr]
    if res is not None:
        in_specs.append(pl.BlockSpec((block_rows, out_dim), lambda i: (i, 0)))
        operands.append(res)

    out = pl.pallas_call(
        functools.partial(_mlp2_body, n_in, act, ln is not None,
                          res is not None, len(widths)),
        grid=grid,
        in_specs=in_specs,
        out_specs=[pl.BlockSpec((block_rows, w), lambda i: (i, 0))
                   for w in widths],
        out_shape=[jax.ShapeDtypeStruct((rows, w), jnp.float32)
                   for w in widths],
    )(*operands)
    return out[0] if out_split is None else out


def _ln(x, g, b):
    m = jnp.mean(x, axis=-1, keepdims=True)
    v = jnp.mean((x - m) ** 2, axis=-1, keepdims=True)
    return (x - m) * lax.rsqrt(v + 1e-5) * g + b


def _mm(xs, ws, b1, w2, b2, act):
    acc = b1
    for x, w in zip(xs, ws):
        acc = acc + jnp.dot(x, w, preferred_element_type=jnp.float32)
    y = act(acc)
    return jnp.dot(y, w2, preferred_element_type=jnp.float32) + b2


def _wspecs(arrs):
    return [pl.BlockSpec(a.shape, lambda i, nd=a.ndim: (0,) * nd)
            for a in arrs]


def _enc_msg_body(refs_in, refs_out):
    (es, c0d, w1, b1, w2, b2, g, bb, m1b, mb1, m2, mb2) = refs_in
    e_out, m0_out, m1_out = refs_out
    e = _ln(_mm([es[...]], [w1[...]], b1[...], w2[...], b2[...], _lrelu),
            g[...], bb[...])
    y = _lrelu(c0d[...] + jnp.dot(e, m1b[...],
                                  preferred_element_type=jnp.float32)
               + mb1[...])
    m = jnp.dot(y, m2[...], preferred_element_type=jnp.float32) + mb2[...]
    e_out[...] = e
    m0_out[...] = m[:, :ED]
    m1_out[...] = m[:, ED:]


def _eupd_msg_body(refs_in, refs_out):
    (ta, tb, e, ec, be1, e2, be2, m1b, mb1, m2, mb2) = refs_in
    e_out, m0_out, m1_out = refs_out
    tbv = tb[...]
    t = _lrelu(ta[...] + tbv[:, :ED]
               + jnp.dot(e[...], ec[...],
                         preferred_element_type=jnp.float32) + be1[...])
    en = jnp.dot(t, e2[...], preferred_element_type=jnp.float32) \
        + be2[...] + e[...]
    y = _lrelu(tbv[:, ED:] + jnp.dot(en, m1b[...],
                                     preferred_element_type=jnp.float32)
               + mb1[...])
    m = jnp.dot(y, m2[...], preferred_element_type=jnp.float32) + mb2[...]
    e_out[...] = en
    m0_out[...] = m[:, :ED]
    m1_out[...] = m[:, ED:]


def _eupd_cls_body(refs_in, refs_out):
    (ta, tb, e, ec, be1, e2, be2, c1c, cb1, c2, cb2) = refs_in
    (p_out,) = refs_out
    tav = ta[...]
    tbv = tb[...]
    t = _lrelu(tav[:, :ED] + tbv[:, :ED]
               + jnp.dot(e[...], ec[...],
                         preferred_element_type=jnp.float32) + be1[...])
    en = jnp.dot(t, e2[...], preferred_element_type=jnp.float32) \
        + be2[...] + e[...]
    y = jnp.maximum(tav[:, ED:ED + 64] + tbv[:, ED:ED + 64]
                    + jnp.dot(en, c1c[...],
                              preferred_element_type=jnp.float32)
                    + cb1[...], 0.0)
    p = jnp.dot(y, c2[...], preferred_element_type=jnp.float32) + cb2[...]
    p_out[...] = p


def _proj_body(nw, *refs):
    h_ref = refs[0]
    ws = refs[1:1 + nw]
    outs = refs[1 + nw:]
    hv = h_ref[...]
    prods = [jnp.dot(hv, w[...], preferred_element_type=jnp.float32)
             for w in ws]
    i = 0
    for o_ref in outs:
        w = 0
        parts = []
        while w < o_ref.shape[1]:
            parts.append(prods[i])
            w += prods[i].shape[1]
            i += 1
        o_ref[...] = parts[0] if len(parts) == 1 else jnp.concatenate(
            parts, axis=1)


def _proj(h, ws, out_widths):
    return pl.pallas_call(
        functools.partial(_proj_body, len(ws)),
        grid=(N // _BN,),
        in_specs=[pl.BlockSpec((_BN, HD), lambda i: (i, 0))] + _wspecs(ws),
        out_specs=[pl.BlockSpec((_BN, w), lambda i: (i, 0))
                   for w in out_widths],
        out_shape=[jax.ShapeDtypeStruct((N, w), jnp.float32)
                   for w in out_widths],
    )(h, *ws)


def _edge_call(body, xs, weights, out_widths):
    n_x = len(xs)
    rows = xs[0].shape[0]
    assert rows % _BE == 0

    def wrapped(*refs):
        body(refs[:n_x + len(weights)], refs[n_x + len(weights):])

    in_specs = [pl.BlockSpec((_BE, x.shape[1]), lambda i: (i, 0))
                for x in xs] + _wspecs(weights)
    return pl.pallas_call(
        wrapped,
        grid=(rows // _BE,),
        in_specs=in_specs,
        out_specs=[pl.BlockSpec((_BE, w), lambda i: (i, 0))
                   for w in out_widths],
        out_shape=[jax.ShapeDtypeStruct((rows, w), jnp.float32)
                   for w in out_widths],
    )(*xs, *weights)


def _r2(b):
    return b.reshape(1, -1)


def kernel(node_visuals, edge_index, edge_spatials, params):
    src = edge_index[0]
    dst = edge_index[1]

    ne = params["node_enc"]
    h = _mlp2([node_visuals], [ne["l1"]["W"]], ne["l1"]["b"],
              ne["l2"]["W"], ne["l2"]["b"], ln=(ne["ln_g"], ne["ln_b"]),
              block_rows=_BN)

    ee = params["edge_enc"]
    es_pad = jnp.pad(edge_spatials, ((0, 0), (0, 5)))
    w1_pad = jnp.pad(ee["l1"]["W"], ((0, 5), (0, 0)))

    srcr = src.reshape(_NS, (E // _NS) // _SC_CH, _SC_CH)
    zeros = jnp.zeros((_SC_CH, ED), jnp.float32)

    layers = params["layers"]
    c1 = params["cls1"]

    def msg_tail_w(lp):
        return [lp["msg1"]["W"][HD:], _r2(lp["msg1"]["b"]),
                lp["msg2"]["W"], _r2(lp["msg2"]["b"])]

    def eupd_tail_w(lp):
        return [lp["e1"]["W"][2 * HD:], _r2(lp["e1"]["b"]),
                lp["e2"]["W"], _r2(lp["e2"]["b"])]

    enc_w = [w1_pad, _r2(ee["l1"]["b"]), ee["l2"]["W"], _r2(ee["l2"]["b"]),
             _r2(ee["ln_g"]), _r2(ee["ln_b"])]
    cls_tail_w = [c1["W"][2 * HD:], _r2(c1["b"]),
                  params["cls2"]["W"], _r2(params["cls2"]["b"])]

    (c0,) = _proj(h, [layers[0]["msg1"]["W"][:HD]], (HD,))
    (c0d,) = _sc_gather([(c0, dst)])
    e, msg0, msg1 = _edge_call(
        _enc_msg_body, [es_pad, c0d],
        enc_w + msg_tail_w(layers[0]), (ED, ED, ED))

    for li, lp in enumerate(layers):
        agg0, agg1 = _sc_scatter_add(msg0, msg1, srcr, zeros)
        agg0, agg1 = agg0[:N], agg1[:N]
        uw = lp["upd1"]["W"]
        h = _mlp2([h, agg0, agg1],
                  [uw[:HD], uw[HD:HD + ED], uw[HD + ED:]],
                  lp["upd1"]["b"], lp["upd2"]["W"], lp["upd2"]["b"],
                  res=h, block_rows=_BN)
        ew = lp["e1"]["W"]
        if li < len(layers) - 1:
            ta, tb = _proj(h, [ew[:HD], ew[HD:2 * HD],
                               layers[li + 1]["msg1"]["W"][:HD]],
                           (ED, ED + HD))
            tas, tbd = _sc_gather([(ta, src), (tb, dst)])
            e, msg0, msg1 = _edge_call(
                _eupd_msg_body, [tas, tbd, e],
                eupd_tail_w(lp) + msg_tail_w(layers[li + 1]), (ED, ED, ED))
        else:
            c1a = jnp.pad(c1["W"][:HD], ((0, 0), (0, 64)))
            c1b = jnp.pad(c1["W"][HD:2 * HD], ((0, 0), (0, 64)))
            ta, tb = _proj(h, [ew[:HD], c1a, ew[HD:2 * HD], c1b],
                           (ED + ED, ED + ED))
            tas, tbd = _sc_gather([(ta, src), (tb, dst)])
            (probs,) = _edge_call(
                _eupd_cls_body, [tas, tbd, e],
                eupd_tail_w(lp) + cls_tail_w, (1,))
    return probs


# speedup vs baseline: 1.0116x; 1.0116x over previous
import functools

import jax
import jax.numpy as jnp
from jax import lax
from jax.experimental import pallas as pl
from jax.experimental.pallas import tpu as pltpu
from jax.experimental.pallas import tpu_sc as plsc

N = 10000
E = 160000
HD = 256
ED = 128

_BN = 2000
_BE = 1280
_EA = 81920
_EB = E - _EA

_NC, _NS = 2, 16
_NW = _NC * _NS
_NP = 10240
_GC = 40
_GNB = 4
_SC_CH = 80


def _sc_mesh():
    return plsc.VectorSubcoreMesh(core_axis_name="c", subcore_axis_name="s")


def _gather_body(nidx, per_w, nch, tail, nb_, *refs):
    tbls = refs[:nidx]
    idx_hbms = refs[nidx:2 * nidx]
    outs = refs[2 * nidx:3 * nidx]
    sc = refs[3 * nidx:]
    nb = nb_
    idx_vs = sc[:nidx]
    bufs = sc[nidx:nidx + nb * nidx]
    gsems = sc[nidx + nb * nidx:nidx + 2 * nb * nidx]
    wsems = sc[nidx + 2 * nb * nidx:nidx + 3 * nb * nidx]

    wid = lax.axis_index("s") * _NC + lax.axis_index("c")
    base = pl.multiple_of(wid * per_w, 8)

    for a in range(nidx):
        pltpu.sync_copy(idx_hbms[a].at[pl.ds(base, per_w)], idx_vs[a])

    def g_start(a, ch, b, sz=_GC):
        off = pl.multiple_of(ch * _GC, 8)
        pltpu.async_copy(tbls[a].at[idx_vs[a].at[pl.ds(off, sz)]],
                         bufs[nb * a + b].at[pl.ds(0, sz)],
                         gsems[nb * a + b])

    def g_wait(a, b, sz=_GC):
        pltpu.make_async_copy(tbls[a].at[idx_vs[a].at[pl.ds(0, sz)]],
                              bufs[nb * a + b].at[pl.ds(0, sz)],
                              gsems[nb * a + b]).wait()

    def w_start(a, ch, b, sz=_GC):
        pltpu.async_copy(bufs[nb * a + b].at[pl.ds(0, sz)],
                         outs[a].at[pl.ds(base + ch * _GC, sz)],
                         wsems[nb * a + b])

    def w_wait(a, b, sz=_GC):
        pltpu.make_async_copy(bufs[nb * a + b].at[pl.ds(0, sz)],
                              outs[a].at[pl.ds(0, sz)],
                              wsems[nb * a + b]).wait()

    lead = min(nb - 1, max(1, nch - 1))

    def step(c2, b, static):
        bg = (b + lead) % nb
        for a in range(nidx):
            if static:
                if c2 + lead < nch:
                    if c2 + lead - nb >= 0:
                        w_wait(a, bg)
                    g_start(a, c2 + lead, bg)
            else:
                @pl.when(c2 + lead < nch)
                def _():
                    @pl.when(c2 + lead - nb >= 0)
                    def _():
                        w_wait(a, bg)
                    g_start(a, c2 + lead, bg)
        for a in range(nidx):
            g_wait(a, b)
            w_start(a, c2, b)

    for a in range(nidx):
        for b in range(lead):
            g_start(a, b, b)

    nfull = (nch // nb) * nb

    @pl.loop(0, nfull, step=nb)
    def _(ch0):
        for b in range(nb):
            step(ch0 + b, b, static=False)

    for c2 in range(nfull, nch):
        step(c2, c2 % nb, static=True)

    for c2 in range(max(0, nch - nb), nch):
        for a in range(nidx):
            w_wait(a, c2 % nb)

    if tail:
        for a in range(nidx):
            g_start(a, nch, 0, tail)
        for a in range(nidx):
            g_wait(a, 0, tail)
            w_start(a, nch, 0, tail)
        for a in range(nidx):
            w_wait(a, 0, tail)


def _sc_gather(pairs):
    nidx = len(pairs)
    el = pairs[0][1].shape[0]
    per_w = el // _NW
    assert per_w * _NW == el and per_w % 8 == 0
    nch = per_w // _GC
    tail = per_w % _GC
    assert tail % 8 == 0
    nb = 8 if nidx == 1 else 4
    scratch = []
    scratch += [pltpu.VMEM((per_w,), jnp.int32) for _ in range(nidx)]
    for t, _ in pairs:
        scratch += [pltpu.VMEM((_GC, t.shape[1]), jnp.float32)
                    for _ in range(nb)]
    scratch += [pltpu.SemaphoreType.DMA for _ in range(2 * nb * nidx)]
    fn = pl.kernel(
        functools.partial(_gather_body, nidx, per_w, nch, tail, nb),
        out_type=tuple(jax.ShapeDtypeStruct((el, t.shape[1]), jnp.float32)
                       for t, _ in pairs),
        mesh=_sc_mesh(),
        scratch_types=scratch,
    )
    return fn(*[t for t, _ in pairs], *[i for _, i in pairs])


def _scatter_pipe(nch, sid, msg_hbm, out_hbm, shared, idx_v, mb, lsems,
                  ssems):
    rows0 = pl.multiple_of(sid * (_NP // _NS), 8)
    ebase = sid * (nch * _SC_CH)

    for j in range(8):
        pltpu.sync_copy(mb[0], shared.at[pl.ds(rows0 + j * _SC_CH, _SC_CH)])
    plsc.subcore_barrier()

    nb = len(mb)
    lead = 1

    def l_start(ch, b):
        pltpu.async_copy(msg_hbm.at[pl.ds(ebase + ch * _SC_CH, _SC_CH)],
                         mb[b], lsems[b])

    def l_wait(b):
        pltpu.make_async_copy(msg_hbm.at[pl.ds(0, _SC_CH)], mb[b],
                              lsems[b]).wait()

    def s_start(ch, b):
        pltpu.async_copy(mb[b], shared.at[idx_v.at[ch]], ssems[b], add=True)

    def s_wait(b):
        pltpu.make_async_copy(mb[b], shared.at[idx_v.at[0]],
                              ssems[b]).wait()

    def step(c2, b, static):
        bg = (b + lead) % nb
        if static:
            if c2 + lead < nch:
                if c2 + lead - nb >= 0:
                    s_wait(bg)
                l_start(c2 + lead, bg)
        else:
            @pl.when(c2 + lead < nch)
            def _():
                @pl.when(c2 + lead - nb >= 0)
                def _():
                    s_wait(bg)
                l_start(c2 + lead, bg)
        l_wait(b)
        s_start(c2, b)

    for b in range(lead):
        l_start(b, b)

    nfull = (nch // nb) * nb

    @pl.loop(0, nfull, step=nb)
    def _(ch0):
        for b in range(nb):
            step(ch0 + b, b, static=False)

    for c2 in range(nfull, nch):
        step(c2, c2 % nb, static=True)

    for c2 in range(max(0, nch - nb), nch):
        s_wait(c2 % nb)

    plsc.subcore_barrier()

    for j in range(8):
        b = j % 2
        pltpu.sync_copy(shared.at[pl.ds(rows0 + j * _SC_CH, _SC_CH)], mb[b])
        pltpu.sync_copy(mb[b], out_hbm.at[pl.ds(rows0 + j * _SC_CH, _SC_CH)])


def _scatter_body(nch, msg0, msg1, srcr, zeros_hbm, out0, out1,
                  shared, idx_v, mb0, mb1, mb2,
                  lsem0, lsem1, lsem2, ssem0, ssem1, ssem2):
    cid = lax.axis_index("c")
    sid = lax.axis_index("s")
    pltpu.sync_copy(srcr.at[sid], idx_v)
    pltpu.sync_copy(zeros_hbm, mb0)

    @pl.when(cid == 0)
    def _():
        _scatter_pipe(nch, sid, msg0, out0, shared, idx_v, (mb0, mb1, mb2),
                      (lsem0, lsem1, lsem2), (ssem0, ssem1, ssem2))

    @pl.when(cid == 1)
    def _():
        _scatter_pipe(nch, sid, msg1, out1, shared, idx_v, (mb0, mb1, mb2),
                      (lsem0, lsem1, lsem2), (ssem0, ssem1, ssem2))


def _sc_scatter_add(msg0, msg1, srcr, zeros):
    nch = srcr.shape[1]
    fn = pl.kernel(
        functools.partial(_scatter_body, nch),
        out_type=(jax.ShapeDtypeStruct((_NP, ED), jnp.float32),
                  jax.ShapeDtypeStruct((_NP, ED), jnp.float32)),
        mesh=_sc_mesh(),
        scratch_types=[
            pltpu.VMEM_SHARED((_NP, ED), jnp.float32),
            pltpu.VMEM((nch, _SC_CH), jnp.int32),
            pltpu.VMEM((_SC_CH, ED), jnp.float32),
            pltpu.VMEM((_SC_CH, ED), jnp.float32),
            pltpu.VMEM((_SC_CH, ED), jnp.float32),
            pltpu.SemaphoreType.DMA,
            pltpu.SemaphoreType.DMA,
            pltpu.SemaphoreType.DMA,
            pltpu.SemaphoreType.DMA,
            pltpu.SemaphoreType.DMA,
            pltpu.SemaphoreType.DMA,
        ],
    )
    return fn(msg0, msg1, srcr, zeros)


def _lrelu(x):
    return jnp.where(x > 0, x, 0.2 * x)


def _mlp2_body(n_in, act, ln, residual, n_proj, *refs):
    xs = refs[:n_in]
    ws = refs[n_in:2 * n_in]
    b1 = refs[2 * n_in]
    w2 = refs[2 * n_in + 1]
    b2 = refs[2 * n_in + 2]
    k = 2 * n_in + 3
    if ln:
        g_ref, bb_ref = refs[k], refs[k + 1]
        k += 2
    if residual:
        res_ref = refs[k]
        k += 1
    pws = refs[k:k + n_proj]
    k += n_proj
    out_ref = refs[k]
    pouts = refs[k + 1:]

    acc = b1[...].astype(jnp.float32)
    for x_ref, w_ref in zip(xs, ws):
        acc = acc + jnp.dot(x_ref[...], w_ref[...],
                            preferred_element_type=jnp.float32)
    y = act(acc)
    out = jnp.dot(y, w2[...], preferred_element_type=jnp.float32) + b2[...]
    if ln:
        m = jnp.mean(out, axis=-1, keepdims=True)
        v = jnp.mean((out - m) ** 2, axis=-1, keepdims=True)
        out = (out - m) * lax.rsqrt(v + 1e-5) * g_ref[...] + bb_ref[...]
    if residual:
        out = out + res_ref[...]
    out_ref[...] = out
    prods = [jnp.dot(out, w[...], preferred_element_type=jnp.float32)
             for w in pws]
    i = 0
    for o_ref in pouts:
        w = 0
        parts = []
        while w < o_ref.shape[1]:
            parts.append(prods[i])
            w += prods[i].shape[1]
            i += 1
        o_ref[...] = parts[0] if len(parts) == 1 else jnp.concatenate(
            parts, axis=1)


def _mlp2(xs, w1s, b1, w2, b2, *, act=_lrelu, ln=None, res=None,
          block_rows=_BE, projs=None):
    rows = xs[0].shape[0]
    assert rows % block_rows == 0
    out_dim = w2.shape[1]
    n_in = len(xs)
    grid = (rows // block_rows,)
    pws, pwidths = projs if projs is not None else ((), ())
    widths = (out_dim,) + tuple(pwidths)

    in_specs = [pl.BlockSpec((block_rows, x.shape[1]), lambda i: (i, 0))
                for x in xs]
    in_specs += [pl.BlockSpec(w.shape, lambda i: (0, 0)) for w in w1s]
    operands = list(xs) + list(w1s)
    b1r = b1.reshape(1, -1)
    b2r = b2.reshape(1, -1)
    in_specs += [pl.BlockSpec(b1r.shape, lambda i: (0, 0)),
                 pl.BlockSpec(w2.shape, lambda i: (0, 0)),
                 pl.BlockSpec(b2r.shape, lambda i: (0, 0))]
    operands += [b1r, w2, b2r]
    if ln is not None:
        g, bb = ln
        gr, bbr = g.reshape(1, -1), bb.reshape(1, -1)
        in_specs += [pl.BlockSpec(gr.shape, lambda i: (0, 0)),
                     pl.BlockSpec(bbr.shape, lambda i: (0, 0))]
        operands += [gr, bbr]
    if res is not None:
        in_specs.append(pl.BlockSpec((block_rows, out_dim), lambda i: (i, 0)))
        operands.append(res)
    in_specs += _wspecs(pws)
    operands += list(pws)

    out = pl.pallas_call(
        functools.partial(_mlp2_body, n_in, act, ln is not None,
                          res is not None, len(pws)),
        grid=grid,
        in_specs=in_specs,
        out_specs=[pl.BlockSpec((block_rows, w), lambda i: (i, 0))
                   for w in widths],
        out_shape=[jax.ShapeDtypeStruct((rows, w), jnp.float32)
                   for w in widths],
    )(*operands)
    return out[0] if projs is None else out


def _ln(x, g, b):
    m = jnp.mean(x, axis=-1, keepdims=True)
    v = jnp.mean((x - m) ** 2, axis=-1, keepdims=True)
    return (x - m) * lax.rsqrt(v + 1e-5) * g + b


def _mm(xs, ws, b1, w2, b2, act):
    acc = b1
    for x, w in zip(xs, ws):
        acc = acc + jnp.dot(x, w, preferred_element_type=jnp.float32)
    y = act(acc)
    return jnp.dot(y, w2, preferred_element_type=jnp.float32) + b2


def _wspecs(arrs):
    return [pl.BlockSpec(a.shape, lambda i, nd=a.ndim: (0,) * nd)
            for a in arrs]


def _enc_msg_body(refs_in, refs_out):
    (es, c0d, w1, b1, w2, b2, g, bb, m1b, mb1, m2, mb2) = refs_in
    e_out, m0_out, m1_out = refs_out
    e = _ln(_mm([es[...]], [w1[...]], b1[...], w2[...], b2[...], _lrelu),
            g[...], bb[...])
    y = _lrelu(c0d[...] + jnp.dot(e, m1b[...],
                                  preferred_element_type=jnp.float32)
               + mb1[...])
    m = jnp.dot(y, m2[...], preferred_element_type=jnp.float32) + mb2[...]
    e_out[...] = e
    m0_out[...] = m[:, :ED]
    m1_out[...] = m[:, ED:]


def _eupd_msg_body(refs_in, refs_out):
    (ta, tb, e, ec, be1, e2, be2, m1b, mb1, m2, mb2) = refs_in
    e_out, m0_out, m1_out = refs_out
    tbv = tb[...]
    t = _lrelu(ta[...] + tbv[:, :ED]
               + jnp.dot(e[...], ec[...],
                         preferred_element_type=jnp.float32) + be1[...])
    en = jnp.dot(t, e2[...], preferred_element_type=jnp.float32) \
        + be2[...] + e[...]
    y = _lrelu(tbv[:, ED:] + jnp.dot(en, m1b[...],
                                     preferred_element_type=jnp.float32)
               + mb1[...])
    m = jnp.dot(y, m2[...], preferred_element_type=jnp.float32) + mb2[...]
    e_out[...] = en
    m0_out[...] = m[:, :ED]
    m1_out[...] = m[:, ED:]


def _eupd_cls_body(refs_in, refs_out):
    (ta, tb, e, ec, be1, e2, be2, c1c, cb1, c2, cb2) = refs_in
    (p_out,) = refs_out
    tav = ta[...]
    tbv = tb[...]
    t = _lrelu(tav[:, :ED] + tbv[:, :ED]
               + jnp.dot(e[...], ec[...],
                         preferred_element_type=jnp.float32) + be1[...])
    en = jnp.dot(t, e2[...], preferred_element_type=jnp.float32) \
        + be2[...] + e[...]
    y = jnp.maximum(tav[:, ED:ED + 64] + tbv[:, ED:ED + 64]
                    + jnp.dot(en, c1c[...],
                              preferred_element_type=jnp.float32)
                    + cb1[...], 0.0)
    p = jnp.dot(y, c2[...], preferred_element_type=jnp.float32) + cb2[...]
    p_out[...] = p


def _proj_body(nw, *refs):
    h_ref = refs[0]
    ws = refs[1:1 + nw]
    outs = refs[1 + nw:]
    hv = h_ref[...]
    prods = [jnp.dot(hv, w[...], preferred_element_type=jnp.float32)
             for w in ws]
    i = 0
    for o_ref in outs:
        w = 0
        parts = []
        while w < o_ref.shape[1]:
            parts.append(prods[i])
            w += prods[i].shape[1]
            i += 1
        o_ref[...] = parts[0] if len(parts) == 1 else jnp.concatenate(
            parts, axis=1)


def _proj(h, ws, out_widths):
    return pl.pallas_call(
        functools.partial(_proj_body, len(ws)),
        grid=(N // _BN,),
        in_specs=[pl.BlockSpec((_BN, HD), lambda i: (i, 0))] + _wspecs(ws),
        out_specs=[pl.BlockSpec((_BN, w), lambda i: (i, 0))
                   for w in out_widths],
        out_shape=[jax.ShapeDtypeStruct((N, w), jnp.float32)
                   for w in out_widths],
    )(h, *ws)


def _edge_call(body, xs, weights, out_widths):
    n_x = len(xs)
    rows = xs[0].shape[0]
    assert rows % _BE == 0

    def wrapped(*refs):
        body(refs[:n_x + len(weights)], refs[n_x + len(weights):])

    in_specs = [pl.BlockSpec((_BE, x.shape[1]), lambda i: (i, 0))
                for x in xs] + _wspecs(weights)
    return pl.pallas_call(
        wrapped,
        grid=(rows // _BE,),
        in_specs=in_specs,
        out_specs=[pl.BlockSpec((_BE, w), lambda i: (i, 0))
                   for w in out_widths],
        out_shape=[jax.ShapeDtypeStruct((rows, w), jnp.float32)
                   for w in out_widths],
    )(*xs, *weights)


def _r2(b):
    return b.reshape(1, -1)


def kernel(node_visuals, edge_index, edge_spatials, params):
    src = edge_index[0]
    dst = edge_index[1]

    ne = params["node_enc"]
    h, c0 = _mlp2([node_visuals], [ne["l1"]["W"]], ne["l1"]["b"],
                  ne["l2"]["W"], ne["l2"]["b"], ln=(ne["ln_g"], ne["ln_b"]),
                  block_rows=_BN,
                  projs=([params["layers"][0]["msg1"]["W"][:HD]], (HD,)))

    ee = params["edge_enc"]
    es_pad = jnp.pad(edge_spatials, ((0, 0), (0, 5)))
    w1_pad = jnp.pad(ee["l1"]["W"], ((0, 5), (0, 0)))

    srcr = src.reshape(_NS, (E // _NS) // _SC_CH, _SC_CH)
    zeros = jnp.zeros((_SC_CH, ED), jnp.float32)

    layers = params["layers"]
    c1 = params["cls1"]

    def msg_tail_w(lp):
        return [lp["msg1"]["W"][HD:], _r2(lp["msg1"]["b"]),
                lp["msg2"]["W"], _r2(lp["msg2"]["b"])]

    def eupd_tail_w(lp):
        return [lp["e1"]["W"][2 * HD:], _r2(lp["e1"]["b"]),
                lp["e2"]["W"], _r2(lp["e2"]["b"])]

    enc_w = [w1_pad, _r2(ee["l1"]["b"]), ee["l2"]["W"], _r2(ee["l2"]["b"]),
             _r2(ee["ln_g"]), _r2(ee["ln_b"])]
    cls_tail_w = [c1["W"][2 * HD:], _r2(c1["b"]),
                  params["cls2"]["W"], _r2(params["cls2"]["b"])]

    (c0d,) = _sc_gather([(c0, dst)])
    e, msg0, msg1 = _edge_call(
        _enc_msg_body, [es_pad, c0d],
        enc_w + msg_tail_w(layers[0]), (ED, ED, ED))

    for li, lp in enumerate(layers):
        agg0, agg1 = _sc_scatter_add(msg0, msg1, srcr, zeros)
        agg0, agg1 = agg0[:N], agg1[:N]
        uw = lp["upd1"]["W"]
        ew = lp["e1"]["W"]
        if li < len(layers) - 1:
            projs = ([ew[:HD], ew[HD:2 * HD],
                      layers[li + 1]["msg1"]["W"][:HD]], (ED, ED + HD))
        else:
            c1a = jnp.pad(c1["W"][:HD], ((0, 0), (0, 64)))
            c1b = jnp.pad(c1["W"][HD:2 * HD], ((0, 0), (0, 64)))
            projs = ([ew[:HD], c1a, ew[HD:2 * HD], c1b],
                     (ED + ED, ED + ED))
        h, ta, tb = _mlp2([h, agg0, agg1],
                          [uw[:HD], uw[HD:HD + ED], uw[HD + ED:]],
                          lp["upd1"]["b"], lp["upd2"]["W"], lp["upd2"]["b"],
                          res=h, block_rows=_BN, projs=projs)
        tas, tbd = _sc_gather([(ta, src), (tb, dst)])
        if li < len(layers) - 1:
            e, msg0, msg1 = _edge_call(
                _eupd_msg_body, [tas, tbd, e],
                eupd_tail_w(lp) + msg_tail_w(layers[li + 1]), (ED, ED, ED))
        else:
            (probs,) = _edge_call(
                _eupd_cls_body, [tas, tbd, e],
                eupd_tail_w(lp) + cls_tail_w, (1,))
    return probs


# speedup vs baseline: 1.0116x; 1.0000x over previous
import functools

import jax
import jax.numpy as jnp
from jax import lax
from jax.experimental import pallas as pl
from jax.experimental.pallas import tpu as pltpu
from jax.experimental.pallas import tpu_sc as plsc

N = 10000
E = 160000
HD = 256
ED = 128

_BN = 2000
_BE = 1280

_NC, _NS = 2, 16
_NW = _NC * _NS
_NP = 10240
_GC = 40
_SC_CH = 80


def _sc_mesh():
    return plsc.VectorSubcoreMesh(core_axis_name="c", subcore_axis_name="s")


def _gather_body(nidx, per_w, nch, tail, nb_, *refs):
    tbls = refs[:nidx]
    idx_hbms = refs[nidx:2 * nidx]
    outs = refs[2 * nidx:3 * nidx]
    sc = refs[3 * nidx:]
    nb = nb_
    idx_vs = sc[:nidx]
    bufs = sc[nidx:nidx + nb * nidx]
    gsems = sc[nidx + nb * nidx:nidx + 2 * nb * nidx]
    wsems = sc[nidx + 2 * nb * nidx:nidx + 3 * nb * nidx]

    wid = lax.axis_index("s") * _NC + lax.axis_index("c")
    base = pl.multiple_of(wid * per_w, 8)

    for a in range(nidx):
        pltpu.sync_copy(idx_hbms[a].at[pl.ds(base, per_w)], idx_vs[a])

    def g_start(a, ch, b, sz=_GC):
        off = pl.multiple_of(ch * _GC, 8)
        pltpu.async_copy(tbls[a].at[idx_vs[a].at[pl.ds(off, sz)]],
                         bufs[nb * a + b].at[pl.ds(0, sz)],
                         gsems[nb * a + b])

    def g_wait(a, b, sz=_GC):
        pltpu.make_async_copy(tbls[a].at[idx_vs[a].at[pl.ds(0, sz)]],
                              bufs[nb * a + b].at[pl.ds(0, sz)],
                              gsems[nb * a + b]).wait()

    def w_start(a, ch, b, sz=_GC):
        pltpu.async_copy(bufs[nb * a + b].at[pl.ds(0, sz)],
                         outs[a].at[pl.ds(base + ch * _GC, sz)],
                         wsems[nb * a + b])

    def w_wait(a, b, sz=_GC):
        pltpu.make_async_copy(bufs[nb * a + b].at[pl.ds(0, sz)],
                              outs[a].at[pl.ds(0, sz)],
                              wsems[nb * a + b]).wait()

    lead = min(nb - 1, max(1, nch - 1))

    def step(c2, b, static):
        bg = (b + lead) % nb
        for a in range(nidx):
            if static:
                if c2 + lead < nch:
                    if c2 + lead - nb >= 0:
                        w_wait(a, bg)
                    g_start(a, c2 + lead, bg)
            else:
                @pl.when(c2 + lead < nch)
                def _():
                    @pl.when(c2 + lead - nb >= 0)
                    def _():
                        w_wait(a, bg)
                    g_start(a, c2 + lead, bg)
        for a in range(nidx):
            g_wait(a, b)
            w_start(a, c2, b)

    for a in range(nidx):
        for b in range(lead):
            g_start(a, b, b)

    nfull = (nch // nb) * nb

    @pl.loop(0, nfull, step=nb)
    def _(ch0):
        for b in range(nb):
            step(ch0 + b, b, static=False)

    for c2 in range(nfull, nch):
        step(c2, c2 % nb, static=True)

    for c2 in range(max(0, nch - nb), nch):
        for a in range(nidx):
            w_wait(a, c2 % nb)

    if tail:
        for a in range(nidx):
            g_start(a, nch, 0, tail)
        for a in range(nidx):
            g_wait(a, 0, tail)
            w_start(a, nch, 0, tail)
        for a in range(nidx):
            w_wait(a, 0, tail)


def _sc_gather(pairs):
    nidx = len(pairs)
    el = pairs[0][1].shape[0]
    per_w = el // _NW
    assert per_w * _NW == el and per_w % 8 == 0
    nch = per_w // _GC
    tail = per_w % _GC
    assert tail % 8 == 0
    nb = 8 if nidx == 1 else 4
    scratch = []
    scratch += [pltpu.VMEM((per_w,), jnp.int32) for _ in range(nidx)]
    for t, _ in pairs:
        scratch += [pltpu.VMEM((_GC, t.shape[1]), jnp.float32)
                    for _ in range(nb)]
    scratch += [pltpu.SemaphoreType.DMA for _ in range(2 * nb * nidx)]
    fn = pl.kernel(
        functools.partial(_gather_body, nidx, per_w, nch, tail, nb),
        out_type=tuple(jax.ShapeDtypeStruct((el, t.shape[1]), jnp.float32)
                       for t, _ in pairs),
        mesh=_sc_mesh(),
        scratch_types=scratch,
    )
    return fn(*[t for t, _ in pairs], *[i for _, i in pairs])


def _scatter_pipe(nch, sid, msg_hbm, out_hbm, shared, idx_v, mb, lsems,
                  ssems):
    rows0 = pl.multiple_of(sid * (_NP // _NS), 8)
    ebase = sid * (nch * _SC_CH)

    for j in range(8):
        pltpu.sync_copy(mb[0], shared.at[pl.ds(rows0 + j * _SC_CH, _SC_CH)])
    plsc.subcore_barrier()

    nb = len(mb)
    lead = 1

    def l_start(ch, b):
        pltpu.async_copy(msg_hbm.at[pl.ds(ebase + ch * _SC_CH, _SC_CH)],
                         mb[b], lsems[b])

    def l_wait(b):
        pltpu.make_async_copy(msg_hbm.at[pl.ds(0, _SC_CH)], mb[b],
                              lsems[b]).wait()

    def s_start(ch, b):
        pltpu.async_copy(mb[b], shared.at[idx_v.at[ch]], ssems[b], add=True)

    def s_wait(b):
        pltpu.make_async_copy(mb[b], shared.at[idx_v.at[0]],
                              ssems[b]).wait()

    def step(c2, b, static):
        bg = (b + lead) % nb
        if static:
            if c2 + lead < nch:
                if c2 + lead - nb >= 0:
                    s_wait(bg)
                l_start(c2 + lead, bg)
        else:
            @pl.when(c2 + lead < nch)
            def _():
                @pl.when(c2 + lead - nb >= 0)
                def _():
                    s_wait(bg)
                l_start(c2 + lead, bg)
        l_wait(b)
        s_start(c2, b)

    for b in range(lead):
        l_start(b, b)

    nfull = (nch // nb) * nb

    @pl.loop(0, nfull, step=nb)
    def _(ch0):
        for b in range(nb):
            step(ch0 + b, b, static=False)

    for c2 in range(nfull, nch):
        step(c2, c2 % nb, static=True)

    for c2 in range(max(0, nch - nb), nch):
        s_wait(c2 % nb)

    plsc.subcore_barrier()

    for j in range(8):
        b = j % 2
        pltpu.sync_copy(shared.at[pl.ds(rows0 + j * _SC_CH, _SC_CH)], mb[b])
        pltpu.sync_copy(mb[b], out_hbm.at[pl.ds(rows0 + j * _SC_CH, _SC_CH)])


def _scatter_body(nch, msg0, msg1, srcr, zeros_hbm, out0, out1,
                  shared, idx_v, mb0, mb1, mb2,
                  lsem0, lsem1, lsem2, ssem0, ssem1, ssem2):
    cid = lax.axis_index("c")
    sid = lax.axis_index("s")
    pltpu.sync_copy(srcr.at[sid], idx_v)
    pltpu.sync_copy(zeros_hbm, mb0)

    @pl.when(cid == 0)
    def _():
        _scatter_pipe(nch, sid, msg0, out0, shared, idx_v, (mb0, mb1, mb2),
                      (lsem0, lsem1, lsem2), (ssem0, ssem1, ssem2))

    @pl.when(cid == 1)
    def _():
        _scatter_pipe(nch, sid, msg1, out1, shared, idx_v, (mb0, mb1, mb2),
                      (lsem0, lsem1, lsem2), (ssem0, ssem1, ssem2))


def _sc_scatter_add(msg0, msg1, srcr, zeros):
    nch = srcr.shape[1]
    fn = pl.kernel(
        functools.partial(_scatter_body, nch),
        out_type=(jax.ShapeDtypeStruct((_NP, ED), jnp.float32),
                  jax.ShapeDtypeStruct((_NP, ED), jnp.float32)),
        mesh=_sc_mesh(),
        scratch_types=[
            pltpu.VMEM_SHARED((_NP, ED), jnp.float32),
            pltpu.VMEM((nch, _SC_CH), jnp.int32),
            pltpu.VMEM((_SC_CH, ED), jnp.float32),
            pltpu.VMEM((_SC_CH, ED), jnp.float32),
            pltpu.VMEM((_SC_CH, ED), jnp.float32),
            pltpu.SemaphoreType.DMA,
            pltpu.SemaphoreType.DMA,
            pltpu.SemaphoreType.DMA,
            pltpu.SemaphoreType.DMA,
            pltpu.SemaphoreType.DMA,
            pltpu.SemaphoreType.DMA,
        ],
    )
    return fn(msg0, msg1, srcr, zeros)


def _lrelu(x):
    return jnp.where(x > 0, x, 0.2 * x)


def _mlp2_body(n_in, act, ln, residual, n_proj, *refs):
    xs = refs[:n_in]
    ws = refs[n_in:2 * n_in]
    b1 = refs[2 * n_in]
    w2 = refs[2 * n_in + 1]
    b2 = refs[2 * n_in + 2]
    k = 2 * n_in + 3
    if ln:
        g_ref, bb_ref = refs[k], refs[k + 1]
        k += 2
    if residual:
        res_ref = refs[k]
        k += 1
    pws = refs[k:k + n_proj]
    k += n_proj
    out_ref = refs[k]
    pouts = refs[k + 1:]

    acc = b1[...].astype(jnp.float32)
    for x_ref, w_ref in zip(xs, ws):
        acc = acc + jnp.dot(x_ref[...], w_ref[...],
                            preferred_element_type=jnp.float32)
    y = act(acc)
    out = jnp.dot(y, w2[...], preferred_element_type=jnp.float32) + b2[...]
    if ln:
        m = jnp.mean(out, axis=-1, keepdims=True)
        v = jnp.mean((out - m) ** 2, axis=-1, keepdims=True)
        out = (out - m) * lax.rsqrt(v + 1e-5) * g_ref[...] + bb_ref[...]
    if residual:
        out = out + res_ref[...]
    out_ref[...] = out
    prods = [jnp.dot(out, w[...], preferred_element_type=jnp.float32)
             for w in pws]
    i = 0
    for o_ref in pouts:
        w = 0
        parts = []
        while w < o_ref.shape[1]:
            parts.append(prods[i])
            w += prods[i].shape[1]
            i += 1
        o_ref[...] = parts[0] if len(parts) == 1 else jnp.concatenate(
            parts, axis=1)


def _mlp2(xs, w1s, b1, w2, b2, *, act=_lrelu, ln=None, res=None,
          block_rows=_BE, projs=None):
    rows = xs[0].shape[0]
    assert rows % block_rows == 0
    out_dim = w2.shape[1]
    n_in = len(xs)
    grid = (rows // block_rows,)
    pws, pwidths = projs if projs is not None else ((), ())
    widths = (out_dim,) + tuple(pwidths)

    in_specs = [pl.BlockSpec((block_rows, x.shape[1]), lambda i: (i, 0))
                for x in xs]
    in_specs += [pl.BlockSpec(w.shape, lambda i: (0, 0)) for w in w1s]
    operands = list(xs) + list(w1s)
    b1r = b1.reshape(1, -1)
    b2r = b2.reshape(1, -1)
    in_specs += [pl.BlockSpec(b1r.shape, lambda i: (0, 0)),
                 pl.BlockSpec(w2.shape, lambda i: (0, 0)),
                 pl.BlockSpec(b2r.shape, lambda i: (0, 0))]
    operands += [b1r, w2, b2r]
    if ln is not None:
        g, bb = ln
        gr, bbr = g.reshape(1, -1), bb.reshape(1, -1)
        in_specs += [pl.BlockSpec(gr.shape, lambda i: (0, 0)),
                     pl.BlockSpec(bbr.shape, lambda i: (0, 0))]
        operands += [gr, bbr]
    if res is not None:
        in_specs.append(pl.BlockSpec((block_rows, out_dim), lambda i: (i, 0)))
        operands.append(res)
    in_specs += _wspecs(pws)
    operands += list(pws)

    out = pl.pallas_call(
        functools.partial(_mlp2_body, n_in, act, ln is not None,
                          res is not None, len(pws)),
        grid=grid,
        in_specs=in_specs,
        out_specs=[pl.BlockSpec((block_rows, w), lambda i: (i, 0))
                   for w in widths],
        out_shape=[jax.ShapeDtypeStruct((rows, w), jnp.float32)
                   for w in widths],
    )(*operands)
    return out[0] if projs is None else out


def _ln(x, g, b):
    m = jnp.mean(x, axis=-1, keepdims=True)
    v = jnp.mean((x - m) ** 2, axis=-1, keepdims=True)
    return (x - m) * lax.rsqrt(v + 1e-5) * g + b


def _mm(xs, ws, b1, w2, b2, act):
    acc = b1
    for x, w in zip(xs, ws):
        acc = acc + jnp.dot(x, w, preferred_element_type=jnp.float32)
    y = act(acc)
    return jnp.dot(y, w2, preferred_element_type=jnp.float32) + b2


def _wspecs(arrs):
    return [pl.BlockSpec(a.shape, lambda i, nd=a.ndim: (0,) * nd)
            for a in arrs]


def _enc_msg_body(refs_in, refs_out):
    (es, c0d, w1, b1, w2, b2, g, bb, m1b, mb1, m2, mb2) = refs_in
    e_out, m0_out, m1_out = refs_out
    e = _ln(_mm([es[...]], [w1[...]], b1[...], w2[...], b2[...], _lrelu),
            g[...], bb[...])
    y = _lrelu(c0d[...] + jnp.dot(e, m1b[...],
                                  preferred_element_type=jnp.float32)
               + mb1[...])
    m = jnp.dot(y, m2[...], preferred_element_type=jnp.float32) + mb2[...]
    e_out[...] = e
    m0_out[...] = m[:, :ED]
    m1_out[...] = m[:, ED:]


def _eupd_msg_body(refs_in, refs_out):
    (ta, tb, e, ec, be1, e2, be2, m1b, mb1, m2, mb2) = refs_in
    e_out, m0_out, m1_out = refs_out
    tbv = tb[...]
    t = _lrelu(ta[...] + tbv[:, :ED]
               + jnp.dot(e[...], ec[...],
                         preferred_element_type=jnp.float32) + be1[...])
    en = jnp.dot(t, e2[...], preferred_element_type=jnp.float32) \
        + be2[...] + e[...]
    y = _lrelu(tbv[:, ED:] + jnp.dot(en, m1b[...],
                                     preferred_element_type=jnp.float32)
               + mb1[...])
    m = jnp.dot(y, m2[...], preferred_element_type=jnp.float32) + mb2[...]
    e_out[...] = en
    m0_out[...] = m[:, :ED]
    m1_out[...] = m[:, ED:]


def _eupd_cls_body(refs_in, refs_out):
    (ta, tb, e, ec, be1, e2, be2, c1c, cb1, c2, cb2) = refs_in
    (p_out,) = refs_out
    tav = ta[...]
    tbv = tb[...]
    t = _lrelu(tav[:, :ED] + tbv[:, :ED]
               + jnp.dot(e[...], ec[...],
                         preferred_element_type=jnp.float32) + be1[...])
    en = jnp.dot(t, e2[...], preferred_element_type=jnp.float32) \
        + be2[...] + e[...]
    y = jnp.maximum(tav[:, ED:ED + 64] + tbv[:, ED:ED + 64]
                    + jnp.dot(en, c1c[...],
                              preferred_element_type=jnp.float32)
                    + cb1[...], 0.0)
    p = jnp.dot(y, c2[...], preferred_element_type=jnp.float32) + cb2[...]
    p_out[...] = p


def _edge_call(body, xs, weights, out_widths):
    n_x = len(xs)
    rows = xs[0].shape[0]
    assert rows % _BE == 0

    def wrapped(*refs):
        body(refs[:n_x + len(weights)], refs[n_x + len(weights):])

    in_specs = [pl.BlockSpec((_BE, x.shape[1]), lambda i: (i, 0))
                for x in xs] + _wspecs(weights)
    return pl.pallas_call(
        wrapped,
        grid=(rows // _BE,),
        in_specs=in_specs,
        out_specs=[pl.BlockSpec((_BE, w), lambda i: (i, 0))
                   for w in out_widths],
        out_shape=[jax.ShapeDtypeStruct((rows, w), jnp.float32)
                   for w in out_widths],
    )(*xs, *weights)


def _r2(b):
    return b.reshape(1, -1)


def kernel(node_visuals, edge_index, edge_spatials, params):
    src = edge_index[0]
    dst = edge_index[1]

    ne = params["node_enc"]
    h, c0 = _mlp2([node_visuals], [ne["l1"]["W"]], ne["l1"]["b"],
                  ne["l2"]["W"], ne["l2"]["b"], ln=(ne["ln_g"], ne["ln_b"]),
                  block_rows=_BN,
                  projs=([params["layers"][0]["msg1"]["W"][:HD]], (HD,)))

    ee = params["edge_enc"]
    es_pad = jnp.pad(edge_spatials, ((0, 0), (0, 5)))
    w1_pad = jnp.pad(ee["l1"]["W"], ((0, 5), (0, 0)))

    srcr = src.reshape(_NS, (E // _NS) // _SC_CH, _SC_CH)
    zeros = jnp.zeros((_SC_CH, ED), jnp.float32)

    layers = params["layers"]
    c1 = params["cls1"]

    def msg_tail_w(lp):
        return [lp["msg1"]["W"][HD:], _r2(lp["msg1"]["b"]),
                lp["msg2"]["W"], _r2(lp["msg2"]["b"])]

    def eupd_tail_w(lp):
        return [lp["e1"]["W"][2 * HD:], _r2(lp["e1"]["b"]),
                lp["e2"]["W"], _r2(lp["e2"]["b"])]

    enc_w = [w1_pad, _r2(ee["l1"]["b"]), ee["l2"]["W"], _r2(ee["l2"]["b"]),
             _r2(ee["ln_g"]), _r2(ee["ln_b"])]
    cls_tail_w = [c1["W"][2 * HD:], _r2(c1["b"]),
                  params["cls2"]["W"], _r2(params["cls2"]["b"])]

    (c0d,) = _sc_gather([(c0, dst)])
    e, msg0, msg1 = _edge_call(
        _enc_msg_body, [es_pad, c0d],
        enc_w + msg_tail_w(layers[0]), (ED, ED, ED))

    for li, lp in enumerate(layers):
        agg0, agg1 = _sc_scatter_add(msg0, msg1, srcr, zeros)
        agg0, agg1 = agg0[:N], agg1[:N]
        uw = lp["upd1"]["W"]
        ew = lp["e1"]["W"]
        if li < len(layers) - 1:
            projs = ([ew[:HD], ew[HD:2 * HD],
                      layers[li + 1]["msg1"]["W"][:HD]], (ED, ED + HD))
        else:
            c1a = jnp.pad(c1["W"][:HD], ((0, 0), (0, 64)))
            c1b = jnp.pad(c1["W"][HD:2 * HD], ((0, 0), (0, 64)))
            projs = ([ew[:HD], c1a, ew[HD:2 * HD], c1b],
                     (ED + ED, ED + ED))
        h, ta, tb = _mlp2([h, agg0, agg1],
                          [uw[:HD], uw[HD:HD + ED], uw[HD + ED:]],
                          lp["upd1"]["b"], lp["upd2"]["W"], lp["upd2"]["b"],
                          res=h, block_rows=_BN, projs=projs)
        tas, tbd = _sc_gather([(ta, src), (tb, dst)])
        if li < len(layers) - 1:
            e, msg0, msg1 = _edge_call(
                _eupd_msg_body, [tas, tbd, e],
                eupd_tail_w(lp) + msg_tail_w(layers[li + 1]), (ED, ED, ED))
        else:
            (probs,) = _edge_call(
                _eupd_cls_body, [tas, tbd, e],
                eupd_tail_w(lp) + cls_tail_w, (1,))
    return probs
